# baseline (speedup 1.0000x reference)
import functools

import jax
import jax.numpy as jnp
from jax import lax
from jax.experimental import pallas as pl
from jax.experimental.pallas import tpu as pltpu
from jax.experimental.pallas import tpu_sc as plsc

N_NODES = 10000
N_EDGES = 320000
D = 128
L = 16

_NC = 2
_NS = 16
_NW = _NC * _NS

_EW = N_EDGES // _NW
_C = 200
_NCHUNK = _EW // _C


def _mlp_body(x_ref, w1_ref, b1_ref, w2_ref, b2_ref, h_ref):
    g = jnp.dot(x_ref[...], w1_ref[...], preferred_element_type=jnp.float32)
    g = jnp.maximum(g + b1_ref[...], 0.0)
    h = jnp.dot(g, w2_ref[...], preferred_element_type=jnp.float32)
    h_ref[...] = (h + b2_ref[...]).astype(jnp.bfloat16)


def _mlp(x, W1, b1, W2, b2):
    n = x.shape[0]
    bn = 2000
    return pl.pallas_call(
        _mlp_body,
        grid=(n // bn,),
        in_specs=[
            pl.BlockSpec((bn, D), lambda i: (i, 0)),
            pl.BlockSpec((D, D), lambda i: (0, 0)),
            pl.BlockSpec((1, D), lambda i: (0, 0)),
            pl.BlockSpec((D, D), lambda i: (0, 0)),
            pl.BlockSpec((1, D), lambda i: (0, 0)),
        ],
        out_specs=pl.BlockSpec((bn, D), lambda i: (i, 0)),
        out_shape=jax.ShapeDtypeStruct((n, D), jnp.bfloat16),
    )(x, W1, b1.reshape(1, D), W2, b2.reshape(1, D))


def _make_edge_dot():
    mesh = plsc.VectorSubcoreMesh(core_axis_name="c", subcore_axis_name="s")

    buf = lambda shape, dt: pltpu.VMEM(shape, dt)

    @functools.partial(
        pl.kernel,
        mesh=mesh,
        out_type=jax.ShapeDtypeStruct((N_EDGES,), jnp.float32),
        scratch_types=[
            buf((_C,), jnp.int32),
            buf((_C,), jnp.int32),
            buf((_C,), jnp.int32),
            buf((_C,), jnp.int32),
            buf((_C, D), jnp.int32),
            buf((_C, D), jnp.int32),
            buf((_C, D), jnp.int32),
            buf((_C, D), jnp.int32),
            buf((_C * 16 + 16,), jnp.float32),
            buf((_C * 4 + 16,), jnp.float32),
            buf((_C + 16,), jnp.float32),
            pltpu.SemaphoreType.DMA,
            pltpu.SemaphoreType.DMA,
        ],
    )
    def edge_dot(h_hbm, src_hbm, dst_hbm, out_hbm,
                 src_v0, src_v1, dst_v0, dst_v1, u_v0, u_v1, v_v0, v_v1,
                 pbuf, qbuf, obuf, sem_u, sem_v):
        wid = lax.axis_index("s") * _NC + lax.axis_index("c")
        ebase = wid * _EW
        src_v = (src_v0, src_v1)
        dst_v = (dst_v0, dst_v1)
        u_v = (u_v0, u_v1)
        v_v = (v_v0, v_v1)

        def fire(c, slot):
            base = ebase + c * _C
            pltpu.sync_copy(src_hbm.at[pl.ds(base, _C)], src_v[slot])
            pltpu.sync_copy(dst_hbm.at[pl.ds(base, _C)], dst_v[slot])
            pltpu.async_copy(h_hbm.at[src_v[slot]], u_v[slot], sem_u)
            pltpu.async_copy(h_hbm.at[dst_v[slot]], v_v[slot], sem_v)

        def wait(slot):
            pltpu.make_async_copy(
                h_hbm.at[src_v[slot]], u_v[slot], sem_u).wait()
            pltpu.make_async_copy(
                h_hbm.at[dst_v[slot]], v_v[slot], sem_v).wait()

        def compute(c, slot):
            uv = u_v[slot]
            vv = v_v[slot]

            mask = jnp.int32(-65536)

            def partial_body(e, _):
                p = jnp.zeros((L,), jnp.float32)
                for j in range(D // (2 * L)):
                    uw = uv[e, pl.ds(j * L, L)]
                    vw = vv[e, pl.ds(j * L, L)]
                    ul = lax.bitcast_convert_type(uw << 16, jnp.float32)
                    vl = lax.bitcast_convert_type(vw << 16, jnp.float32)
                    uh = lax.bitcast_convert_type(uw & mask, jnp.float32)
                    vh = lax.bitcast_convert_type(vw & mask, jnp.float32)
                    p += ul * vl
                    p += uh * vh
                pbuf[pl.ds(e * 16, L)] = p
                return ()

            lax.fori_loop(0, _C, partial_body, (), unroll=2)

            def fold1(e, _):
                s = (pbuf[pl.ds(e * 16, L)] + pbuf[pl.ds(e * 16 + 4, L)] +
                     pbuf[pl.ds(e * 16 + 8, L)] + pbuf[pl.ds(e * 16 + 12, L)])
                qbuf[pl.ds(e * 4, L)] = s
                return ()

            def fold2(e, _):
                s = (qbuf[pl.ds(e * 4, L)] + qbuf[pl.ds(e * 4 + 1, L)] +
                     qbuf[pl.ds(e * 4 + 2, L)] + qbuf[pl.ds(e * 4 + 3, L)])
                obuf[pl.ds(e, L)] = s
                return ()

            lax.fori_loop(0, _C, fold1, (), unroll=4)
            lax.fori_loop(0, _C, fold2, (), unroll=4)
            pltpu.sync_copy(obuf.at[pl.ds(0, _C)],
                            out_hbm.at[pl.ds(ebase + c * _C, _C)])

        fire(0, 0)

        def pair_body(k, _):
            c0 = k * 2
            fire(c0 + 1, 1)
            wait(0)
            compute(c0, 0)

            @pl.when(c0 + 2 < _NCHUNK)
            def _():
                fire(c0 + 2, 0)

            wait(1)
            compute(c0 + 1, 1)
            return ()

        lax.fori_loop(0, _NCHUNK // 2, pair_body, ())

    return edge_dot


_edge_dot = _make_edge_dot()


def kernel(x, edge_index, W1, b1, W2, b2):
    hb = _mlp(x, W1, b1, W2, b2)
    hw = lax.bitcast_convert_type(hb.reshape(N_NODES, D // 2, 2), jnp.int32)
    hw = jnp.concatenate([hw, jnp.zeros_like(hw)], axis=1)
    src = edge_index[0].astype(jnp.int32)
    dst = edge_index[1].astype(jnp.int32)
    score = _edge_dot(hw, src, dst)
    return score.reshape(N_EDGES, 1)


# speedup vs baseline: 1.2255x; 1.2255x over previous
import functools

import jax
import jax.numpy as jnp
from jax import lax
from jax.experimental import pallas as pl
from jax.experimental.pallas import tpu as pltpu
from jax.experimental.pallas import tpu_sc as plsc

N_NODES = 10000
N_EDGES = 320000
D = 128
L = 16

_NC = 2
_NS = 16
_NW = _NC * _NS

_EW = N_EDGES // _NW
_C = 200
_NCHUNK = _EW // _C


def _mlp_body(x_ref, w1_ref, b1_ref, w2_ref, b2_ref, h_ref):
    g = jnp.dot(x_ref[...], w1_ref[...], preferred_element_type=jnp.float32)
    g = jnp.maximum(g + b1_ref[...], 0.0)
    h = jnp.dot(g, w2_ref[...], preferred_element_type=jnp.float32)
    h_ref[...] = (h + b2_ref[...]).astype(jnp.bfloat16)


def _mlp(x, W1, b1, W2, b2):
    n = x.shape[0]
    bn = 2000
    return pl.pallas_call(
        _mlp_body,
        grid=(n // bn,),
        in_specs=[
            pl.BlockSpec((bn, D), lambda i: (i, 0)),
            pl.BlockSpec((D, D), lambda i: (0, 0)),
            pl.BlockSpec((1, D), lambda i: (0, 0)),
            pl.BlockSpec((D, D), lambda i: (0, 0)),
            pl.BlockSpec((1, D), lambda i: (0, 0)),
        ],
        out_specs=pl.BlockSpec((bn, D), lambda i: (i, 0)),
        out_shape=jax.ShapeDtypeStruct((n, D), jnp.bfloat16),
    )(x, W1, b1.reshape(1, D), W2, b2.reshape(1, D))


def _make_edge_dot():
    mesh = plsc.VectorSubcoreMesh(core_axis_name="c", subcore_axis_name="s")

    buf = lambda shape, dt: pltpu.VMEM(shape, dt)

    @functools.partial(
        pl.kernel,
        mesh=mesh,
        out_type=jax.ShapeDtypeStruct((N_EDGES,), jnp.float32),
        scratch_types=[
            buf((_C,), jnp.int32),
            buf((_C,), jnp.int32),
            buf((_C,), jnp.int32),
            buf((_C,), jnp.int32),
            buf((_C, D), jnp.int32),
            buf((_C, D), jnp.int32),
            buf((_C, D), jnp.int32),
            buf((_C, D), jnp.int32),
            buf((_C * 16 + 16,), jnp.float32),
            buf((_C * 4 + 16,), jnp.float32),
            buf((_C + 16,), jnp.float32),
            pltpu.SemaphoreType.DMA,
            pltpu.SemaphoreType.DMA,
        ],
    )
    def edge_dot(h_hbm, src_hbm, dst_hbm, out_hbm,
                 src_v0, src_v1, dst_v0, dst_v1, u_v0, u_v1, v_v0, v_v1,
                 pbuf, qbuf, obuf, sem_u, sem_v):
        wid = lax.axis_index("s") * _NC + lax.axis_index("c")
        ebase = wid * _EW
        src_v = (src_v0, src_v1)
        dst_v = (dst_v0, dst_v1)
        u_v = (u_v0, u_v1)
        v_v = (v_v0, v_v1)

        def fire(c, slot):
            base = ebase + c * _C
            pltpu.sync_copy(src_hbm.at[pl.ds(base, _C)], src_v[slot])
            pltpu.sync_copy(dst_hbm.at[pl.ds(base, _C)], dst_v[slot])
            pltpu.async_copy(h_hbm.at[src_v[slot]], u_v[slot], sem_u)
            pltpu.async_copy(h_hbm.at[dst_v[slot]], v_v[slot], sem_v)

        def wait(slot):
            pltpu.make_async_copy(
                h_hbm.at[src_v[slot]], u_v[slot], sem_u).wait()
            pltpu.make_async_copy(
                h_hbm.at[dst_v[slot]], v_v[slot], sem_v).wait()

        def compute(c, slot):
            uv = u_v[slot]
            vv = v_v[slot]

            mask = jnp.int32(-65536)

            @plsc.parallel_loop(0, _C, step=1, unroll=4)
            def _(e):
                plo = jnp.zeros((L,), jnp.float32)
                phi = jnp.zeros((L,), jnp.float32)
                for j in range(D // (2 * L)):
                    uw = uv[e, pl.ds(j * L, L)]
                    vw = vv[e, pl.ds(j * L, L)]
                    ul = lax.bitcast_convert_type(uw << 16, jnp.float32)
                    vl = lax.bitcast_convert_type(vw << 16, jnp.float32)
                    uh = lax.bitcast_convert_type(uw & mask, jnp.float32)
                    vh = lax.bitcast_convert_type(vw & mask, jnp.float32)
                    plo += ul * vl
                    phi += uh * vh
                pbuf[pl.ds(e * 16, L)] = plo + phi

            def fold1(e, _):
                s = (pbuf[pl.ds(e * 16, L)] + pbuf[pl.ds(e * 16 + 4, L)] +
                     pbuf[pl.ds(e * 16 + 8, L)] + pbuf[pl.ds(e * 16 + 12, L)])
                qbuf[pl.ds(e * 4, L)] = s
                return ()

            def fold2(e, _):
                s = (qbuf[pl.ds(e * 4, L)] + qbuf[pl.ds(e * 4 + 1, L)] +
                     qbuf[pl.ds(e * 4 + 2, L)] + qbuf[pl.ds(e * 4 + 3, L)])
                obuf[pl.ds(e, L)] = s
                return ()

            lax.fori_loop(0, _C, fold1, (), unroll=4)
            lax.fori_loop(0, _C, fold2, (), unroll=4)
            pltpu.sync_copy(obuf.at[pl.ds(0, _C)],
                            out_hbm.at[pl.ds(ebase + c * _C, _C)])

        fire(0, 0)

        def pair_body(k, _):
            c0 = k * 2
            fire(c0 + 1, 1)
            wait(0)
            compute(c0, 0)

            @pl.when(c0 + 2 < _NCHUNK)
            def _():
                fire(c0 + 2, 0)

            wait(1)
            compute(c0 + 1, 1)
            return ()

        lax.fori_loop(0, _NCHUNK // 2, pair_body, ())

    return edge_dot


_edge_dot = _make_edge_dot()


def kernel(x, edge_index, W1, b1, W2, b2):
    hb = _mlp(x, W1, b1, W2, b2)
    hw = lax.bitcast_convert_type(hb.reshape(N_NODES, D // 2, 2), jnp.int32)
    hw = jnp.concatenate([hw, jnp.zeros_like(hw)], axis=1)
    src = edge_index[0].astype(jnp.int32)
    dst = edge_index[1].astype(jnp.int32)
    score = _edge_dot(hw, src, dst)
    return score.reshape(N_EDGES, 1)
